# Initial kernel scaffold; baseline (speedup 1.0000x reference)
import jax
import jax.numpy as jnp
from jax.experimental import pallas as pl


def kernel(x, up_index, up_attr, boundary_index, boundary_attr, W_mu, b_mu, Wu1, bu1, Wu2, bu2, Wb1, bb1, Wb2, bb2, Wc, bc, gu1, betau1, gu2, betau2, gb1, betab1, gb2, betab2, gc, betac):
    raise NotImplementedError("write your pallas kernel here")


# speedup vs baseline: 1.9258x; 1.9258x over previous
import functools

import jax
import jax.numpy as jnp
from jax import lax
from jax.experimental import pallas as pl
from jax.experimental.pallas import tpu as pltpu
from jax.experimental.pallas import tpu_sc as plsc

N = 10000
E_UP = 160000
E_B = 20000
D = 256
H = 256
NC = 2
NS = 16
CH = 80
ROW_BLK = 640
ROW_LAST = N - 15 * ROW_BLK
UP_CHUNKS = E_UP // CH
UP_CHUNKS_PER_TILE = UP_CHUNKS // NS
B_CHUNKS = E_B // CH


def _p1_body(x_ref, ba_ref, w_ref, b_ref, xa_out, xh_out, bh_out):
    a = jnp.dot(x_ref[...], w_ref[...], preferred_element_type=jnp.float32)
    a = a + b_ref[...]
    xa_out[0] = a[:, :128]
    xa_out[1] = a[:, 128:]
    xh_out[0] = x_ref[:, :128]
    xh_out[1] = x_ref[:, 128:]
    bh_out[0] = ba_ref[:, :128]
    bh_out[1] = ba_ref[:, 128:]


def _run_p1(x, boundary_attr, W_mu, b_mu):
    blk = 1000
    grid = (N // blk,)
    out = pl.pallas_call(
        _p1_body,
        grid=grid,
        in_specs=[
            pl.BlockSpec((blk, D), lambda i: (i, 0)),
            pl.BlockSpec((blk, D), lambda i: (i, 0)),
            pl.BlockSpec((D, D), lambda i: (0, 0)),
            pl.BlockSpec((1, D), lambda i: (0, 0)),
        ],
        out_specs=[
            pl.BlockSpec((2, blk, 128), lambda i: (0, i, 0)),
            pl.BlockSpec((2, blk, 128), lambda i: (0, i, 0)),
            pl.BlockSpec((2, blk, 128), lambda i: (0, i, 0)),
        ],
        out_shape=[jax.ShapeDtypeStruct((2, N, 128), jnp.float32)] * 3,
    )(x, boundary_attr, W_mu[:D], b_mu.reshape(1, D))
    xa, xh, bh = out
    return (xa.reshape(2 * N, 128), xh.reshape(2 * N, 128),
            bh.reshape(2 * N, 128))


def _p2_body(ua_ref, w_ref, t_out):
    r = jnp.dot(ua_ref[...], w_ref[...], preferred_element_type=jnp.float32)
    t_out[0] = r[:, :128]
    t_out[1] = r[:, 128:]


def _run_p2(up_attr, W_mu):
    blk = 2000
    grid = (E_UP // blk,)
    t = pl.pallas_call(
        _p2_body,
        grid=grid,
        in_specs=[
            pl.BlockSpec((blk, D), lambda i: (i, 0)),
            pl.BlockSpec((D, D), lambda i: (1, 0)),
        ],
        out_specs=pl.BlockSpec((2, blk, 128), lambda i: (0, i, 0)),
        out_shape=jax.ShapeDtypeStruct((2, E_UP, 128), jnp.float32),
    )(up_attr, W_mu)
    return t.reshape(2 * E_UP, 128)


def _sc_body(xa, t, xh, ba, usrc, udst, bsrc, bdst, ou, ob,
             acc, tbuf, gbuf, sbuf, dbuf, sem):
    c = lax.axis_index("c")
    s = lax.axis_index("s")
    r0 = s * ROW_BLK
    xoff = c * N
    toff = c * E_UP

    def adjust_src(_r, _):
        sl = pl.ds(_r * 16, 16)
        sbuf[sl] = sbuf[sl] + xoff
        return 0

    def rows_copy(src_ref, dst_ref, soff, doff):
        @pl.when(s < NS - 1)
        def _():
            pltpu.sync_copy(src_ref.at[pl.ds(soff, ROW_BLK)],
                            dst_ref.at[pl.ds(doff, ROW_BLK)])

        @pl.when(s == NS - 1)
        def _():
            pltpu.sync_copy(src_ref.at[pl.ds(soff, ROW_LAST)],
                            dst_ref.at[pl.ds(doff, ROW_LAST)])

    rows_copy(xh, acc, xoff + r0, r0)
    plsc.subcore_barrier()

    def up_chunk(j, _):
        base = (s + j * NS) * CH
        pltpu.sync_copy(usrc.at[pl.ds(base, CH)], sbuf)
        pltpu.sync_copy(udst.at[pl.ds(base, CH)], dbuf)
        pltpu.sync_copy(t.at[pl.ds(toff + base, CH)], tbuf)
        lax.fori_loop(0, CH // 16, adjust_src, 0)
        pltpu.async_copy(xa.at[sbuf], gbuf, sem).wait()

        def rowfn(r, _):
            for k in range(8):
                sl = pl.ds(k * 16, 16)
                tbuf[r, sl] = jnp.maximum(tbuf[r, sl] + gbuf[r, sl], 0.0)
            return 0

        lax.fori_loop(0, CH, rowfn, 0)
        pltpu.sync_copy(tbuf, acc.at[dbuf], add=True)
        return 0

    lax.fori_loop(0, UP_CHUNKS_PER_TILE, up_chunk, 0)
    plsc.subcore_barrier()
    rows_copy(acc, ou, r0, xoff + r0)

    rows_copy(xh, acc, xoff + r0, r0)
    plsc.subcore_barrier()

    nk = (B_CHUNKS - s + NS - 1) // NS

    def b_chunk(j, _):
        base = (s + j * NS) * CH
        pltpu.sync_copy(bsrc.at[pl.ds(base, CH)], sbuf)
        pltpu.sync_copy(bdst.at[pl.ds(base, CH)], dbuf)
        lax.fori_loop(0, CH // 16, adjust_src, 0)
        pltpu.async_copy(ba.at[sbuf], gbuf, sem).wait()
        pltpu.sync_copy(gbuf, acc.at[dbuf], add=True)
        return 0

    lax.fori_loop(0, nk, b_chunk, 0)
    plsc.subcore_barrier()
    rows_copy(acc, ob, r0, xoff + r0)


def _run_sc(xa, t, xh, ba, up_index, boundary_index):
    mesh = plsc.VectorSubcoreMesh(core_axis_name="c", subcore_axis_name="s",
                                  num_cores=NC, num_subcores=NS)
    f = pl.kernel(
        _sc_body,
        out_type=[jax.ShapeDtypeStruct((2 * N, 128), jnp.float32)] * 2,
        mesh=mesh,
        scratch_types=[
            pltpu.VMEM_SHARED((N, 128), jnp.float32),
            pltpu.VMEM((CH, 128), jnp.float32),
            pltpu.VMEM((CH, 128), jnp.float32),
            pltpu.VMEM((CH,), jnp.int32),
            pltpu.VMEM((CH,), jnp.int32),
            pltpu.SemaphoreType.DMA,
        ],
    )
    return f(xa, t, xh, ba, up_index[1], up_index[0],
             boundary_index[0], boundary_index[1])


def _bn_relu(h, g, beta):
    mu = jnp.mean(h, axis=0, keepdims=True)
    d = h - mu
    var = jnp.mean(d * d, axis=0, keepdims=True)
    return jnp.maximum(g * d * lax.rsqrt(var + 1e-5) + beta, 0.0)


def _mlp_body(ou_ref, ob_ref, wu1, bu1, wu2, bu2, wb1, bb1, wb2, bb2,
              wc, bc, gu1, betau1, gu2, betau2, gb1, betab1, gb2, betab2,
              gc, betac, out_ref):
    dot = functools.partial(jnp.dot, preferred_element_type=jnp.float32)
    u = dot(ou_ref[:N], wu1[:128]) + dot(ou_ref[N:], wu1[128:]) + bu1[...]
    u = _bn_relu(u, gu1[...], betau1[...])
    u = _bn_relu(dot(u, wu2[...]) + bu2[...], gu2[...], betau2[...])
    b = dot(ob_ref[:N], wb1[:128]) + dot(ob_ref[N:], wb1[128:]) + bb1[...]
    b = _bn_relu(b, gb1[...], betab1[...])
    b = _bn_relu(dot(b, wb2[...]) + bb2[...], gb2[...], betab2[...])
    o = dot(u, wc[:H]) + dot(b, wc[H:]) + bc[...]
    out_ref[...] = _bn_relu(o, gc[...], betac[...])


def _run_mlp(ou, ob, Wu1, bu1, Wu2, bu2, Wb1, bb1, Wb2, bb2, Wc, bc,
             gu1, betau1, gu2, betau2, gb1, betab1, gb2, betab2, gc, betac):
    row = lambda v: v.reshape(1, H)
    return pl.pallas_call(
        _mlp_body,
        out_shape=jax.ShapeDtypeStruct((N, H), jnp.float32),
    )(ou, ob, Wu1, row(bu1), Wu2, row(bu2), Wb1, row(bb1), Wb2, row(bb2),
      Wc, row(bc), row(gu1), row(betau1), row(gu2), row(betau2),
      row(gb1), row(betab1), row(gb2), row(betab2), row(gc), row(betac))


def kernel(x, up_index, up_attr, boundary_index, boundary_attr, W_mu, b_mu,
           Wu1, bu1, Wu2, bu2, Wb1, bb1, Wb2, bb2, Wc, bc, gu1, betau1,
           gu2, betau2, gb1, betab1, gb2, betab2, gc, betac):
    xa, xh, bh = _run_p1(x, boundary_attr, W_mu, b_mu)
    t = _run_p2(up_attr, W_mu)
    ou, ob = _run_sc(xa, t, xh, bh, up_index, boundary_index)
    return _run_mlp(ou, ob, Wu1, bu1, Wu2, bu2, Wb1, bb1, Wb2, bb2, Wc, bc,
                    gu1, betau1, gu2, betau2, gb1, betab1, gb2, betab2,
                    gc, betac)


# speedup vs baseline: 2.8529x; 1.4814x over previous
import functools

import jax
import jax.numpy as jnp
from jax import lax
from jax.experimental import pallas as pl
from jax.experimental.pallas import tpu as pltpu
from jax.experimental.pallas import tpu_sc as plsc

N = 10000
E_UP = 160000
E_B = 20000
D = 256
H = 256
NC = 2
NS = 16
CH = 64
ROW_BLK = 640
ROW_LAST = N - 15 * ROW_BLK
EPT = E_UP // NS
UP_CHUNKS_PER_TILE = EPT // CH
UP_REM = EPT - UP_CHUNKS_PER_TILE * CH
CHB = 80
B_CHUNKS = E_B // CHB


def _p1_body(x_ref, ba_ref, w_ref, b_ref, xa_out, xh_out, bh_out):
    a = jnp.dot(x_ref[...], w_ref[...], preferred_element_type=jnp.float32)
    a = a + b_ref[...]
    xa_out[0] = a[:, :128]
    xa_out[1] = a[:, 128:]
    xh_out[0] = x_ref[:, :128]
    xh_out[1] = x_ref[:, 128:]
    bh_out[0] = ba_ref[:, :128]
    bh_out[1] = ba_ref[:, 128:]


def _run_p1(x, boundary_attr, W_mu, b_mu):
    blk = 1000
    grid = (N // blk,)
    out = pl.pallas_call(
        _p1_body,
        grid=grid,
        in_specs=[
            pl.BlockSpec((blk, D), lambda i: (i, 0)),
            pl.BlockSpec((blk, D), lambda i: (i, 0)),
            pl.BlockSpec((D, D), lambda i: (0, 0)),
            pl.BlockSpec((1, D), lambda i: (0, 0)),
        ],
        out_specs=[
            pl.BlockSpec((2, blk, 128), lambda i: (0, i, 0)),
            pl.BlockSpec((2, blk, 128), lambda i: (0, i, 0)),
            pl.BlockSpec((2, blk, 128), lambda i: (0, i, 0)),
        ],
        out_shape=[jax.ShapeDtypeStruct((2, N, 128), jnp.float32)] * 3,
    )(x, boundary_attr, W_mu[:D], b_mu.reshape(1, D))
    xa, xh, bh = out
    return (xa.reshape(2 * N, 128), xh.reshape(2 * N, 128),
            bh.reshape(2 * N, 128))


def _p2_body(ua_ref, w_ref, t_out):
    r = jnp.dot(ua_ref[...], w_ref[...], preferred_element_type=jnp.float32)
    t_out[0] = r[:, :128]
    t_out[1] = r[:, 128:]


def _run_p2(up_attr, W_mu):
    blk = 2000
    grid = (E_UP // blk,)
    t = pl.pallas_call(
        _p2_body,
        grid=grid,
        in_specs=[
            pl.BlockSpec((blk, D), lambda i: (i, 0)),
            pl.BlockSpec((D, D), lambda i: (1, 0)),
        ],
        out_specs=pl.BlockSpec((2, blk, 128), lambda i: (0, i, 0)),
        out_shape=jax.ShapeDtypeStruct((2, E_UP, 128), jnp.float32),
    )(up_attr, W_mu)
    return t.reshape(2 * E_UP, 128)


def _sc_body(xa, t, xh, ba, usrc2, udst, bsrc2, bdst, ou, ob,
             acc, t0, t1, g0, g1, s0, s1, d0, d1,
             te, ge, se, de, gb_b, sb_b, db_b,
             semL0, semL1, semG0, semG1, semB):
    c = lax.axis_index("c")
    s = lax.axis_index("s")
    r0 = s * ROW_BLK
    xoff = c * N
    toff = c * E_UP
    ebase = s * EPT

    tb = (t0, t1)
    gb = (g0, g1)
    sb = (s0, s1)
    db = (d0, d1)
    semL = (semL0, semL1)
    semG = (semG0, semG1)

    def rows_copy(src_ref, dst_ref, soff, doff):
        @pl.when(s < NS - 1)
        def _():
            pltpu.sync_copy(src_ref.at[pl.ds(soff, ROW_BLK)],
                            dst_ref.at[pl.ds(doff, ROW_BLK)])

        @pl.when(s == NS - 1)
        def _():
            pltpu.sync_copy(src_ref.at[pl.ds(soff, ROW_LAST)],
                            dst_ref.at[pl.ds(doff, ROW_LAST)])

    rows_copy(xh, acc, xoff + r0, r0)
    plsc.subcore_barrier()

    def issue_loads(b, j):
        base = ebase + j * CH
        pltpu.async_copy(t.at[pl.ds(toff + base, CH)], tb[b], semL[b])
        pltpu.async_copy(usrc2.at[pl.ds(toff + base, CH)], sb[b], semL[b])
        pltpu.async_copy(udst.at[pl.ds(base, CH)], db[b], semL[b])

    def wait_loads(b):
        pltpu.make_async_copy(t.at[pl.ds(toff + ebase, CH)], tb[b],
                              semL[b]).wait()
        pltpu.make_async_copy(usrc2.at[pl.ds(toff + ebase, CH)], sb[b],
                              semL[b]).wait()
        pltpu.make_async_copy(udst.at[pl.ds(ebase, CH)], db[b],
                              semL[b]).wait()

    def issue_gather(b):
        pltpu.async_copy(xa.at[sb[b]], gb[b], semG[b])

    def wait_gather(b):
        pltpu.make_async_copy(xa.at[sb[b]], gb[b], semG[b]).wait()

    def compute_scatter(b):
        _tb, _gb = tb[b], gb[b]

        @plsc.parallel_loop(0, CH, unroll=2)
        def _(r):
            for k in range(8):
                sl = pl.ds(k * 16, 16)
                _tb[r, sl] = jnp.maximum(_tb[r, sl] + _gb[r, sl], 0.0)

        pltpu.sync_copy(_tb, acc.at[db[b]], add=True)

    issue_loads(0, 0)
    wait_loads(0)
    issue_gather(0)
    issue_loads(1, 1)

    NP = UP_CHUNKS_PER_TILE // 2

    def pair(p, _):
        wait_gather(0)
        wait_loads(1)
        issue_gather(1)
        compute_scatter(0)

        @pl.when(p < NP - 1)
        def _():
            issue_loads(0, 2 * p + 2)

        wait_gather(1)
        compute_scatter(1)

        @pl.when(p < NP - 1)
        def _():
            wait_loads(0)
            issue_gather(0)
            issue_loads(1, 2 * p + 3)

        return 0

    lax.fori_loop(0, NP, pair, 0)

    rbase = ebase + UP_CHUNKS_PER_TILE * CH
    pltpu.sync_copy(t.at[pl.ds(toff + rbase, UP_REM)], te)
    pltpu.sync_copy(usrc2.at[pl.ds(toff + rbase, UP_REM)], se)
    pltpu.sync_copy(udst.at[pl.ds(rbase, UP_REM)], de)
    pltpu.async_copy(xa.at[se], ge, semB).wait()

    @plsc.parallel_loop(0, UP_REM, unroll=2)
    def _(r):
        for k in range(8):
            sl = pl.ds(k * 16, 16)
            te[r, sl] = jnp.maximum(te[r, sl] + ge[r, sl], 0.0)

    pltpu.sync_copy(te, acc.at[de], add=True)

    plsc.subcore_barrier()
    rows_copy(acc, ou, r0, xoff + r0)

    rows_copy(xh, acc, xoff + r0, r0)
    plsc.subcore_barrier()

    nk = (B_CHUNKS - s + NS - 1) // NS

    def b_chunk(j, _):
        base = (s + j * NS) * CHB
        pltpu.sync_copy(bsrc2.at[pl.ds(c * E_B + base, CHB)], sb_b)
        pltpu.sync_copy(bdst.at[pl.ds(base, CHB)], db_b)
        pltpu.async_copy(ba.at[sb_b], gb_b, semB).wait()
        pltpu.sync_copy(gb_b, acc.at[db_b], add=True)
        return 0

    lax.fori_loop(0, nk, b_chunk, 0)
    plsc.subcore_barrier()
    rows_copy(acc, ob, r0, xoff + r0)


def _run_sc(xa, t, xh, ba, up_index, boundary_index):
    mesh = plsc.VectorSubcoreMesh(core_axis_name="c", subcore_axis_name="s",
                                  num_cores=NC, num_subcores=NS)
    f = pl.kernel(
        _sc_body,
        out_type=[jax.ShapeDtypeStruct((2 * N, 128), jnp.float32)] * 2,
        mesh=mesh,
        scratch_types=[
            pltpu.VMEM_SHARED((N, 128), jnp.float32),
            pltpu.VMEM((CH, 128), jnp.float32),
            pltpu.VMEM((CH, 128), jnp.float32),
            pltpu.VMEM((CH, 128), jnp.float32),
            pltpu.VMEM((CH, 128), jnp.float32),
            pltpu.VMEM((CH,), jnp.int32),
            pltpu.VMEM((CH,), jnp.int32),
            pltpu.VMEM((CH,), jnp.int32),
            pltpu.VMEM((CH,), jnp.int32),
            pltpu.VMEM((UP_REM, 128), jnp.float32),
            pltpu.VMEM((UP_REM, 128), jnp.float32),
            pltpu.VMEM((UP_REM,), jnp.int32),
            pltpu.VMEM((UP_REM,), jnp.int32),
            pltpu.VMEM((CHB, 128), jnp.float32),
            pltpu.VMEM((CHB,), jnp.int32),
            pltpu.VMEM((CHB,), jnp.int32),
            pltpu.SemaphoreType.DMA,
            pltpu.SemaphoreType.DMA,
            pltpu.SemaphoreType.DMA,
            pltpu.SemaphoreType.DMA,
            pltpu.SemaphoreType.DMA,
        ],
    )
    usrc2 = jnp.concatenate([up_index[1], up_index[1] + N])
    bsrc2 = jnp.concatenate([boundary_index[0], boundary_index[0] + N])
    return f(xa, t, xh, ba, usrc2, up_index[0], bsrc2, boundary_index[1])


def _bn_relu(h, g, beta):
    mu = jnp.mean(h, axis=0, keepdims=True)
    d = h - mu
    var = jnp.mean(d * d, axis=0, keepdims=True)
    return jnp.maximum(g * d * lax.rsqrt(var + 1e-5) + beta, 0.0)


def _mlp_body(ou_ref, ob_ref, wu1, bu1, wu2, bu2, wb1, bb1, wb2, bb2,
              wc, bc, gu1, betau1, gu2, betau2, gb1, betab1, gb2, betab2,
              gc, betac, out_ref):
    dot = functools.partial(jnp.dot, preferred_element_type=jnp.float32)
    u = dot(ou_ref[:N], wu1[:128]) + dot(ou_ref[N:], wu1[128:]) + bu1[...]
    u = _bn_relu(u, gu1[...], betau1[...])
    u = _bn_relu(dot(u, wu2[...]) + bu2[...], gu2[...], betau2[...])
    b = dot(ob_ref[:N], wb1[:128]) + dot(ob_ref[N:], wb1[128:]) + bb1[...]
    b = _bn_relu(b, gb1[...], betab1[...])
    b = _bn_relu(dot(b, wb2[...]) + bb2[...], gb2[...], betab2[...])
    o = dot(u, wc[:H]) + dot(b, wc[H:]) + bc[...]
    out_ref[...] = _bn_relu(o, gc[...], betac[...])


def _run_mlp(ou, ob, Wu1, bu1, Wu2, bu2, Wb1, bb1, Wb2, bb2, Wc, bc,
             gu1, betau1, gu2, betau2, gb1, betab1, gb2, betab2, gc, betac):
    row = lambda v: v.reshape(1, H)
    return pl.pallas_call(
        _mlp_body,
        out_shape=jax.ShapeDtypeStruct((N, H), jnp.float32),
    )(ou, ob, Wu1, row(bu1), Wu2, row(bu2), Wb1, row(bb1), Wb2, row(bb2),
      Wc, row(bc), row(gu1), row(betau1), row(gu2), row(betau2),
      row(gb1), row(betab1), row(gb2), row(betab2), row(gc), row(betac))


def kernel(x, up_index, up_attr, boundary_index, boundary_attr, W_mu, b_mu,
           Wu1, bu1, Wu2, bu2, Wb1, bb1, Wb2, bb2, Wc, bc, gu1, betau1,
           gu2, betau2, gb1, betab1, gb2, betab2, gc, betac):
    xa, xh, bh = _run_p1(x, boundary_attr, W_mu, b_mu)
    t = _run_p2(up_attr, W_mu)
    ou, ob = _run_sc(xa, t, xh, bh, up_index, boundary_index)
    return _run_mlp(ou, ob, Wu1, bu1, Wu2, bu2, Wb1, bb1, Wb2, bb2, Wc, bc,
                    gu1, betau1, gu2, betau2, gb1, betab1, gb2, betab2,
                    gc, betac)


# speedup vs baseline: 2.8566x; 1.0013x over previous
import functools

import jax
import jax.numpy as jnp
from jax import lax
from jax.experimental import pallas as pl
from jax.experimental.pallas import tpu as pltpu
from jax.experimental.pallas import tpu_sc as plsc

N = 10000
E_UP = 160000
E_B = 20000
D = 256
H = 256
NC = 2
NS = 16
CH = 64
ROW_BLK = 640
ROW_LAST = N - 15 * ROW_BLK
EPT = E_UP // NS
UP_CHUNKS_PER_TILE = EPT // CH
UP_REM = EPT - UP_CHUNKS_PER_TILE * CH
CHB = 80
B_CHUNKS = E_B // CHB


def _p1_body(x_ref, ba_ref, w_ref, b_ref, xa_out, xh_out, bh_out):
    a = jnp.dot(x_ref[...], w_ref[...], preferred_element_type=jnp.float32)
    a = a + b_ref[...]
    xa_out[0] = a[:, :128]
    xa_out[1] = a[:, 128:]
    xh_out[0] = x_ref[:, :128]
    xh_out[1] = x_ref[:, 128:]
    bh_out[0] = ba_ref[:, :128]
    bh_out[1] = ba_ref[:, 128:]


def _run_p1(x, boundary_attr, W_mu, b_mu):
    blk = 1000
    grid = (N // blk,)
    out = pl.pallas_call(
        _p1_body,
        grid=grid,
        in_specs=[
            pl.BlockSpec((blk, D), lambda i: (i, 0)),
            pl.BlockSpec((blk, D), lambda i: (i, 0)),
            pl.BlockSpec((D, D), lambda i: (0, 0)),
            pl.BlockSpec((1, D), lambda i: (0, 0)),
        ],
        out_specs=[
            pl.BlockSpec((2, blk, 128), lambda i: (0, i, 0)),
            pl.BlockSpec((2, blk, 128), lambda i: (0, i, 0)),
            pl.BlockSpec((2, blk, 128), lambda i: (0, i, 0)),
        ],
        out_shape=[jax.ShapeDtypeStruct((2, N, 128), jnp.float32)] * 3,
    )(x, boundary_attr, W_mu[:D], b_mu.reshape(1, D))
    xa, xh, bh = out
    return (xa.reshape(2 * N, 128), xh.reshape(2 * N, 128),
            bh.reshape(2 * N, 128))


def _p2_body(ua_ref, w_ref, t_out):
    r = jnp.dot(ua_ref[...], w_ref[...], preferred_element_type=jnp.float32)
    t_out[0] = r[:, :128]
    t_out[1] = r[:, 128:]


def _run_p2(up_attr, W_mu):
    blk = 2000
    grid = (E_UP // blk,)
    t = pl.pallas_call(
        _p2_body,
        grid=grid,
        in_specs=[
            pl.BlockSpec((blk, D), lambda i: (i, 0)),
            pl.BlockSpec((D, D), lambda i: (1, 0)),
        ],
        out_specs=pl.BlockSpec((2, blk, 128), lambda i: (0, i, 0)),
        out_shape=jax.ShapeDtypeStruct((2, E_UP, 128), jnp.float32),
    )(up_attr, W_mu)
    return t.reshape(2 * E_UP, 128)


def _sc_body(xa, t, xh, ba, usrc2, udst, bsrc2, bdst, ou, ob,
             acc, t0, t1, g0, g1, s0, s1, d0, d1,
             te, ge, se, de, gb_b, sb_b, db_b,
             semL0, semL1, semG0, semG1, semB):
    c = lax.axis_index("c")
    s = lax.axis_index("s")
    r0 = s * ROW_BLK
    xoff = c * N
    toff = c * E_UP
    ebase = s * EPT

    tb = (t0, t1)
    gb = (g0, g1)
    sb = (s0, s1)
    db = (d0, d1)
    semL = (semL0, semL1)
    semG = (semG0, semG1)

    def rows_copy(src_ref, dst_ref, soff, doff):
        @pl.when(s < NS - 1)
        def _():
            pltpu.sync_copy(src_ref.at[pl.ds(soff, ROW_BLK)],
                            dst_ref.at[pl.ds(doff, ROW_BLK)])

        @pl.when(s == NS - 1)
        def _():
            pltpu.sync_copy(src_ref.at[pl.ds(soff, ROW_LAST)],
                            dst_ref.at[pl.ds(doff, ROW_LAST)])

    rows_copy(xh, acc, xoff + r0, r0)
    plsc.subcore_barrier()

    def issue_loads(b, j):
        base = ebase + j * CH
        pltpu.async_copy(t.at[pl.ds(toff + base, CH)], tb[b], semL[b])
        pltpu.async_copy(usrc2.at[pl.ds(toff + base, CH)], sb[b], semL[b])
        pltpu.async_copy(udst.at[pl.ds(base, CH)], db[b], semL[b])

    def wait_loads(b):
        pltpu.make_async_copy(t.at[pl.ds(toff + ebase, CH)], tb[b],
                              semL[b]).wait()
        pltpu.make_async_copy(usrc2.at[pl.ds(toff + ebase, CH)], sb[b],
                              semL[b]).wait()
        pltpu.make_async_copy(udst.at[pl.ds(ebase, CH)], db[b],
                              semL[b]).wait()

    def issue_gather(b):
        pltpu.async_copy(xa.at[sb[b]], gb[b], semG[b])

    def wait_gather(b):
        pltpu.make_async_copy(xa.at[sb[b]], gb[b], semG[b]).wait()

    def compute_scatter(b):
        _tb, _gb = tb[b], gb[b]

        @plsc.parallel_loop(0, CH, unroll=4)
        def _(r):
            for k in range(8):
                sl = pl.ds(k * 16, 16)
                _tb[r, sl] = jnp.maximum(_tb[r, sl] + _gb[r, sl], 0.0)

        pltpu.sync_copy(_tb, acc.at[db[b]], add=True)

    issue_loads(0, 0)
    wait_loads(0)
    issue_gather(0)
    issue_loads(1, 1)

    NP = UP_CHUNKS_PER_TILE // 2

    def pair(p, _):
        wait_gather(0)
        wait_loads(1)
        issue_gather(1)
        compute_scatter(0)

        @pl.when(p < NP - 1)
        def _():
            issue_loads(0, 2 * p + 2)

        wait_gather(1)
        compute_scatter(1)

        @pl.when(p < NP - 1)
        def _():
            wait_loads(0)
            issue_gather(0)
            issue_loads(1, 2 * p + 3)

        return 0

    lax.fori_loop(0, NP, pair, 0)

    rbase = ebase + UP_CHUNKS_PER_TILE * CH
    pltpu.sync_copy(t.at[pl.ds(toff + rbase, UP_REM)], te)
    pltpu.sync_copy(usrc2.at[pl.ds(toff + rbase, UP_REM)], se)
    pltpu.sync_copy(udst.at[pl.ds(rbase, UP_REM)], de)
    pltpu.async_copy(xa.at[se], ge, semB).wait()

    @plsc.parallel_loop(0, UP_REM, unroll=2)
    def _(r):
        for k in range(8):
            sl = pl.ds(k * 16, 16)
            te[r, sl] = jnp.maximum(te[r, sl] + ge[r, sl], 0.0)

    pltpu.sync_copy(te, acc.at[de], add=True)

    plsc.subcore_barrier()
    rows_copy(acc, ou, r0, xoff + r0)

    rows_copy(xh, acc, xoff + r0, r0)
    plsc.subcore_barrier()

    nk = (B_CHUNKS - s + NS - 1) // NS

    def b_chunk(j, _):
        base = (s + j * NS) * CHB
        pltpu.sync_copy(bsrc2.at[pl.ds(c * E_B + base, CHB)], sb_b)
        pltpu.sync_copy(bdst.at[pl.ds(base, CHB)], db_b)
        pltpu.async_copy(ba.at[sb_b], gb_b, semB).wait()
        pltpu.sync_copy(gb_b, acc.at[db_b], add=True)
        return 0

    lax.fori_loop(0, nk, b_chunk, 0)
    plsc.subcore_barrier()
    rows_copy(acc, ob, r0, xoff + r0)


def _run_sc(xa, t, xh, ba, up_index, boundary_index):
    mesh = plsc.VectorSubcoreMesh(core_axis_name="c", subcore_axis_name="s",
                                  num_cores=NC, num_subcores=NS)
    f = pl.kernel(
        _sc_body,
        out_type=[jax.ShapeDtypeStruct((2 * N, 128), jnp.float32)] * 2,
        mesh=mesh,
        scratch_types=[
            pltpu.VMEM_SHARED((N, 128), jnp.float32),
            pltpu.VMEM((CH, 128), jnp.float32),
            pltpu.VMEM((CH, 128), jnp.float32),
            pltpu.VMEM((CH, 128), jnp.float32),
            pltpu.VMEM((CH, 128), jnp.float32),
            pltpu.VMEM((CH,), jnp.int32),
            pltpu.VMEM((CH,), jnp.int32),
            pltpu.VMEM((CH,), jnp.int32),
            pltpu.VMEM((CH,), jnp.int32),
            pltpu.VMEM((UP_REM, 128), jnp.float32),
            pltpu.VMEM((UP_REM, 128), jnp.float32),
            pltpu.VMEM((UP_REM,), jnp.int32),
            pltpu.VMEM((UP_REM,), jnp.int32),
            pltpu.VMEM((CHB, 128), jnp.float32),
            pltpu.VMEM((CHB,), jnp.int32),
            pltpu.VMEM((CHB,), jnp.int32),
            pltpu.SemaphoreType.DMA,
            pltpu.SemaphoreType.DMA,
            pltpu.SemaphoreType.DMA,
            pltpu.SemaphoreType.DMA,
            pltpu.SemaphoreType.DMA,
        ],
    )
    usrc2 = jnp.concatenate([up_index[1], up_index[1] + N])
    bsrc2 = jnp.concatenate([boundary_index[0], boundary_index[0] + N])
    return f(xa, t, xh, ba, usrc2, up_index[0], bsrc2, boundary_index[1])


def _bn_relu(h, g, beta):
    mu = jnp.mean(h, axis=0, keepdims=True)
    d = h - mu
    var = jnp.mean(d * d, axis=0, keepdims=True)
    return jnp.maximum(g * d * lax.rsqrt(var + 1e-5) + beta, 0.0)


def _mlp_body(ou_ref, ob_ref, wu1, bu1, wu2, bu2, wb1, bb1, wb2, bb2,
              wc, bc, gu1, betau1, gu2, betau2, gb1, betab1, gb2, betab2,
              gc, betac, out_ref):
    dot = functools.partial(jnp.dot, preferred_element_type=jnp.float32)
    u = dot(ou_ref[:N], wu1[:128]) + dot(ou_ref[N:], wu1[128:]) + bu1[...]
    u = _bn_relu(u, gu1[...], betau1[...])
    u = _bn_relu(dot(u, wu2[...]) + bu2[...], gu2[...], betau2[...])
    b = dot(ob_ref[:N], wb1[:128]) + dot(ob_ref[N:], wb1[128:]) + bb1[...]
    b = _bn_relu(b, gb1[...], betab1[...])
    b = _bn_relu(dot(b, wb2[...]) + bb2[...], gb2[...], betab2[...])
    o = dot(u, wc[:H]) + dot(b, wc[H:]) + bc[...]
    out_ref[...] = _bn_relu(o, gc[...], betac[...])


def _run_mlp(ou, ob, Wu1, bu1, Wu2, bu2, Wb1, bb1, Wb2, bb2, Wc, bc,
             gu1, betau1, gu2, betau2, gb1, betab1, gb2, betab2, gc, betac):
    row = lambda v: v.reshape(1, H)
    return pl.pallas_call(
        _mlp_body,
        out_shape=jax.ShapeDtypeStruct((N, H), jnp.float32),
    )(ou, ob, Wu1, row(bu1), Wu2, row(bu2), Wb1, row(bb1), Wb2, row(bb2),
      Wc, row(bc), row(gu1), row(betau1), row(gu2), row(betau2),
      row(gb1), row(betab1), row(gb2), row(betab2), row(gc), row(betac))


def kernel(x, up_index, up_attr, boundary_index, boundary_attr, W_mu, b_mu,
           Wu1, bu1, Wu2, bu2, Wb1, bb1, Wb2, bb2, Wc, bc, gu1, betau1,
           gu2, betau2, gb1, betab1, gb2, betab2, gc, betac):
    xa, xh, bh = _run_p1(x, boundary_attr, W_mu, b_mu)
    t = _run_p2(up_attr, W_mu)
    ou, ob = _run_sc(xa, t, xh, bh, up_index, boundary_index)
    return _run_mlp(ou, ob, Wu1, bu1, Wu2, bu2, Wb1, bb1, Wb2, bb2, Wc, bc,
                    gu1, betau1, gu2, betau2, gb1, betab1, gb2, betab2,
                    gc, betac)
